# baseline (speedup 1.0000x reference)
import functools

import jax
import jax.numpy as jnp
from jax import lax
from jax.experimental import pallas as pl
from jax.experimental.pallas import tpu as pltpu
from jax.experimental.pallas import tpu_sc as plsc

BATCH = 4096
SEQ = 50
VOCAB = 1000
D = 1000
N = BATCH * SEQ
NSHARD = 8
TAIL = D - 128 * (NSHARD - 1)

NUM_WORKERS = 32
PER_WORKER = N // NUM_WORKERS
CHUNK = 200
NCH = PER_WORKER // CHUNK

_MESH = plsc.VectorSubcoreMesh(core_axis_name="c", subcore_axis_name="s")


@functools.partial(
    pl.kernel,
    out_type=jax.ShapeDtypeStruct((N, D), jnp.float32),
    mesh=_MESH,
    scratch_types=[
        pltpu.VMEM((PER_WORKER,), jnp.int32),
        pltpu.VMEM((2, CHUNK, 128), jnp.float32),
        pltpu.VMEM((CHUNK, TAIL), jnp.float32),
        pltpu.SemaphoreType.DMA((2,)),
        pltpu.SemaphoreType.DMA((2,)),
        pltpu.SemaphoreType.DMA,
    ],
    compiler_params=pltpu.CompilerParams(use_tc_tiling_on_sc=True),
)
def _gather_rows(xf_hbm, *refs):
    shards = refs[:NSHARD]
    out_hbm = refs[NSHARD]
    idx_v, rows, tail_v, sem_g, sem_w, sem_t = refs[NSHARD + 1:]

    wid = lax.axis_index("s") * 2 + lax.axis_index("c")
    base = wid * PER_WORKER

    def idx_slice(k):
        return idx_v.at[pl.ds(pl.multiple_of(k * CHUNK, 8), CHUNK)]

    def out_win(k, c, width):
        return out_hbm.at[pl.ds(pl.multiple_of(base + k * CHUNK, 8), CHUNK)] \
                      .at[:, pl.ds(c * 128, width)]

    def G(k, c):
        return pltpu.make_async_copy(shards[c].at[idx_slice(k)],
                                     rows.at[c % 2], sem_g.at[c % 2])

    def W(k, c):
        return pltpu.make_async_copy(rows.at[c % 2], out_win(k, c, 128),
                                     sem_w.at[c % 2])

    def TW(k):
        return pltpu.make_async_copy(tail_v, out_win(k, NSHARD - 1, TAIL),
                                     sem_t)

    def tail_compact():
        def row(r, cr):
            for off in (0, 16, 32, 48, 64, 80, TAIL - 16):
                tail_v[r, pl.ds(off, 16)] = rows[1, r, pl.ds(off, 16)]
            return cr

        lax.fori_loop(0, CHUNK, row, 0)

    pltpu.sync_copy(xf_hbm.at[pl.ds(base, PER_WORKER)], idx_v)

    G(0, 0).start()
    G(0, 1).start()

    def step_k(k, cr):
        for c in range(NSHARD):
            G(k, c).wait()
            if c < NSHARD - 1:
                W(k, c).start()
                if c < NSHARD - 2:
                    W(k, c).wait()
                    G(k, c + 2).start()
                else:

                    @pl.when(k + 1 < NCH)
                    def _():
                        W(k, c).wait()
                        G(k + 1, 0).start()

            else:

                @pl.when(k >= 1)
                def _():
                    TW(k - 1).wait()

                tail_compact()
                TW(k).start()

                @pl.when(k + 1 < NCH)
                def _():
                    G(k + 1, 1).start()

        return cr

    lax.fori_loop(0, NCH, step_k, 0)
    W(NCH - 1, NSHARD - 2).wait()
    TW(NCH - 1).wait()


def kernel(x, table):
    tp = jnp.pad(table, ((0, 0), (0, NSHARD * 128 - D)))
    shards = tuple(tp[:, c * 128:(c + 1) * 128] for c in range(NSHARD))
    flat = _gather_rows(x.reshape(N), *shards)
    return flat.reshape(BATCH, SEQ, D)
